# baseline (speedup 1.0000x reference)
import functools

import jax
import jax.numpy as jnp
from jax import lax
from jax.experimental import pallas as pl
from jax.experimental.pallas import tpu as pltpu
from jax.experimental.pallas import tpu_sc as plsc

B, W, R, M = 256, 4, 8, 8192
NC, NS, L = 2, 16, 16
NW = NC * NS
ROWS = B // NW

_mesh = plsc.VectorSubcoreMesh(core_axis_name="c", subcore_axis_name="s")


@functools.partial(
    pl.kernel,
    mesh=_mesh,
    out_type=jax.ShapeDtypeStruct((B, M), jnp.float32),
    scratch_types=[
        pltpu.VMEM((W, M), jnp.float32),
        pltpu.VMEM((R, M), jnp.float32),
        pltpu.VMEM((M,), jnp.float32),
        pltpu.VMEM((R, L), jnp.float32),
        pltpu.VMEM((M,), jnp.float32),
    ],
)
def _freeness_sc(ww_hbm, fgb_hbm, rw_hbm, prev_hbm, out_hbm,
                 ww_v, rw_v, prev_v, fgb_v, out_v):
    wid = lax.axis_index("s") * NC + lax.axis_index("c")
    base = wid * ROWS
    for j in range(ROWS):
        b = base + j
        pltpu.sync_copy(ww_hbm.at[b], ww_v)
        pltpu.sync_copy(rw_hbm.at[b], rw_v)
        pltpu.sync_copy(prev_hbm.at[b], prev_v)
        pltpu.sync_copy(fgb_hbm.at[b], fgb_v)
        fg = [fgb_v[r] for r in range(R)]

        def body(i, carry):
            sl = pl.ds(i * L, L)
            prev = prev_v[sl]
            keep = (1.0 - ww_v[0, sl]) * (1.0 - ww_v[1, sl])
            keep = keep * (1.0 - ww_v[2, sl]) * (1.0 - ww_v[3, sl])
            usage = prev + (1.0 - prev) * (1.0 - keep)
            phi = 1.0 - fg[0] * rw_v[0, sl]
            for r in range(1, R):
                phi = phi * (1.0 - fg[r] * rw_v[r, sl])
            out_v[sl] = usage * phi
            return carry

        lax.fori_loop(0, M // L, body, 0)
        pltpu.sync_copy(out_v, out_hbm.at[b])


def kernel(write_weights, free_gate, read_weights, prev_usage):
    fgb = jnp.broadcast_to(free_gate[:, :, None], (B, R, L))
    return _freeness_sc(write_weights, fgb, read_weights, prev_usage)


# speedup vs baseline: 2.1179x; 2.1179x over previous
import functools

import jax
import jax.numpy as jnp
from jax import lax
from jax.experimental import pallas as pl
from jax.experimental.pallas import tpu as pltpu
from jax.experimental.pallas import tpu_sc as plsc

B, W, R, M = 256, 4, 8, 8192
NC, NS, L = 2, 16, 16
NW = NC * NS

F_SC = 64
ROWS = F_SC // NW
B_TC = B - F_SC
BB = 16

_mesh = plsc.VectorSubcoreMesh(core_axis_name="c", subcore_axis_name="s")


@functools.partial(
    pl.kernel,
    mesh=_mesh,
    out_type=jax.ShapeDtypeStruct((F_SC, M), jnp.float32),
    scratch_types=[
        pltpu.VMEM((W, M), jnp.float32),
        pltpu.VMEM((R, M), jnp.float32),
        pltpu.VMEM((M,), jnp.float32),
        pltpu.VMEM((R, L), jnp.float32),
        pltpu.VMEM((M,), jnp.float32),
    ],
)
def _freeness_sc(ww_hbm, fgb_hbm, rw_hbm, prev_hbm, out_hbm,
                 ww_v, rw_v, prev_v, fgb_v, out_v):
    wid = lax.axis_index("s") * NC + lax.axis_index("c")
    base = wid * ROWS
    for j in range(ROWS):
        b = base + j
        pltpu.sync_copy(ww_hbm.at[b], ww_v)
        pltpu.sync_copy(rw_hbm.at[b], rw_v)
        pltpu.sync_copy(prev_hbm.at[b], prev_v)
        pltpu.sync_copy(fgb_hbm.at[b], fgb_v)
        fg = [fgb_v[r] for r in range(R)]

        def body(i, carry):
            sl = pl.ds(i * L, L)
            prev = prev_v[sl]
            keep = (1.0 - ww_v[0, sl]) * (1.0 - ww_v[1, sl])
            keep = keep * (1.0 - ww_v[2, sl]) * (1.0 - ww_v[3, sl])
            usage = prev + (1.0 - prev) * (1.0 - keep)
            phi = 1.0 - fg[0] * rw_v[0, sl]
            for r in range(1, R):
                phi = phi * (1.0 - fg[r] * rw_v[r, sl])
            out_v[sl] = usage * phi
            return carry

        lax.fori_loop(0, M // L, body, 0)
        pltpu.sync_copy(out_v, out_hbm.at[b])


def _freeness_tc(fg_ref, ww_ref, rw_ref, prev_ref, out_ref):
    for b in range(BB):
        prev = prev_ref[b]
        keep = 1.0 - ww_ref[b, 0]
        for w in range(1, W):
            keep = keep * (1.0 - ww_ref[b, w])
        usage = prev + (1.0 - prev) * (1.0 - keep)
        phi = 1.0 - fg_ref[b, 0] * rw_ref[b, 0]
        for r in range(1, R):
            phi = phi * (1.0 - fg_ref[b, r] * rw_ref[b, r])
        out_ref[b] = usage * phi


def kernel(write_weights, free_gate, read_weights, prev_usage):
    fgb = jnp.broadcast_to(free_gate[:F_SC, :, None], (F_SC, R, L))
    out_sc = _freeness_sc(write_weights, fgb, read_weights, prev_usage)

    off = F_SC // BB
    out_tc = pl.pallas_call(
        _freeness_tc,
        grid=(B_TC // BB,),
        in_specs=[
            pl.BlockSpec((BB, R), lambda i: (i + off, 0),
                         memory_space=pltpu.SMEM),
            pl.BlockSpec((BB, W, M), lambda i: (i + off, 0, 0)),
            pl.BlockSpec((BB, R, M), lambda i: (i + off, 0, 0)),
            pl.BlockSpec((BB, M), lambda i: (i + off, 0)),
        ],
        out_specs=pl.BlockSpec((BB, M), lambda i: (i, 0)),
        out_shape=jax.ShapeDtypeStruct((B_TC, M), jnp.float32),
    )(free_gate, write_weights, read_weights, prev_usage)

    return jnp.concatenate([out_sc, out_tc], axis=0)


# speedup vs baseline: 3.6647x; 1.7303x over previous
import jax
import jax.numpy as jnp
from jax.experimental import pallas as pl
from jax.experimental.pallas import tpu as pltpu

B, W, R, M = 256, 4, 8, 8192
BB = 32


def _freeness_kernel(fg_ref, ww_ref, rw_ref, prev_ref, out_ref):
    for b in range(BB):
        prev = prev_ref[b]
        keep = 1.0 - ww_ref[b, 0]
        for w in range(1, W):
            keep = keep * (1.0 - ww_ref[b, w])
        usage = prev + (1.0 - prev) * (1.0 - keep)
        phi = 1.0 - fg_ref[b, 0] * rw_ref[b, 0]
        for r in range(1, R):
            phi = phi * (1.0 - fg_ref[b, r] * rw_ref[b, r])
        out_ref[b] = usage * phi


def kernel(write_weights, free_gate, read_weights, prev_usage):
    grid = (B // BB,)
    return pl.pallas_call(
        _freeness_kernel,
        grid=grid,
        in_specs=[
            pl.BlockSpec((BB, R), lambda i: (i, 0), memory_space=pltpu.SMEM),
            pl.BlockSpec((BB, W, M), lambda i: (i, 0, 0)),
            pl.BlockSpec((BB, R, M), lambda i: (i, 0, 0)),
            pl.BlockSpec((BB, M), lambda i: (i, 0)),
        ],
        out_specs=pl.BlockSpec((BB, M), lambda i: (i, 0)),
        out_shape=jax.ShapeDtypeStruct((B, M), jnp.float32),
    )(free_gate, write_weights, read_weights, prev_usage)
